# baseline (speedup 1.0000x reference)
import functools

import jax
import jax.numpy as jnp
from jax import lax
from jax.experimental import pallas as pl
from jax.experimental.pallas import tpu as pltpu
from jax.experimental.pallas import tpu_sc as plsc

NC = 2
NS = 16
LANES = 128

BR = 2000


def _seg_sum_sc(n_pad, e_groups, feat, gpc, tail_sizes, spmem_table):
  rpt = n_pad // NS
  gfloor = e_groups // NS
  grem = e_groups - NS * gfloor
  period = 2 * gpc
  assert sum(tail_sizes) >= period - 1 and max(tail_sizes) <= gpc

  mesh = plsc.VectorSubcoreMesh(core_axis_name="c", subcore_axis_name="s")

  out_type = [jax.ShapeDtypeStruct((n_pad, 128), jnp.float32)]
  scratch = [
      pltpu.VMEM_SHARED((n_pad, feat), jnp.float32),
      [pltpu.VMEM((2, gpc, LANES), jnp.int32)] * 2,
      [pltpu.VMEM((gpc * LANES, feat), jnp.float32)] * 2,
      [pltpu.SemaphoreType.DMA] * 2,
      [pltpu.SemaphoreType.DMA] * 2,
  ]
  if spmem_table:
    scratch.append(pltpu.VMEM_SHARED((n_pad, feat), jnp.float32))
  else:
    out_type.append(jax.ShapeDtypeStruct((NC, n_pad, feat), jnp.float32))

  @functools.partial(
      pl.kernel,
      out_type=tuple(out_type),
      mesh=mesh,
      scratch_types=scratch,
      compiler_params=pltpu.CompilerParams(use_tc_tiling_on_sc=False),
  )
  def seg_sum(tabx, eg, out, *rest):
    if spmem_table:
      acc, sd_v, rows_v, gsem, ssem, stab = rest
    else:
      tabh, acc, sd_v, rows_v, gsem, ssem = rest
    c = lax.axis_index("c")
    s = lax.axis_index("s")
    r0 = s * rpt
    pltpu.sync_copy(tabx.at[pl.ds(r0, rpt), pl.ds(c * feat, feat)],
                    acc.at[pl.ds(r0, rpt)])
    if spmem_table:
      tab = stab
      pltpu.sync_copy(tabx.at[pl.ds(r0, rpt), pl.ds(c * feat, feat)],
                      stab.at[pl.ds(r0, rpt)])
    else:
      tab = tabh.at[c]
      pltpu.sync_copy(acc.at[pl.ds(r0, rpt)], tabh.at[c, pl.ds(r0, rpt)])
    plsc.subcore_barrier()

    base = s * gfloor + jnp.minimum(s, grem)
    gcnt = gfloor + jnp.where(s < grem, 1, 0)
    npairs = gcnt // period

    def drain(n_groups, sem, p, to_acc):
      nr = n_groups * LANES
      if to_acc:
        pltpu.make_async_copy(rows_v[p].at[pl.ds(0, nr)],
                              acc.at[pl.ds(0, nr)], sem).wait()
      else:
        pltpu.make_async_copy(tab.at[pl.ds(0, nr)],
                              rows_v[p].at[pl.ds(0, nr)], sem).wait()

    def fire_gathers(p, g0):
      pltpu.sync_copy(eg.at[:, pl.ds(g0, gpc)], sd_v[p])
      for j in range(gpc):
        pltpu.async_copy(tab.at[sd_v[p].at[0, j]],
                         rows_v[p].at[pl.ds(j * LANES, LANES)], gsem[p])

    for p in range(2):
      fire_gathers(p, base + p * gpc)

    def pair(k, carry):
      for p in range(2):
        drain(gpc, gsem[p], p, False)
        for j in range(gpc):
          pltpu.async_copy(rows_v[p].at[pl.ds(j * LANES, LANES)],
                           acc.at[sd_v[p].at[1, j]], ssem[p], add=True)

        @pl.when(k < npairs - 1)
        def _():
          drain(gpc, ssem[p], p, True)
          fire_gathers(p, base + (2 * k + p + 2) * gpc)
      return carry

    lax.fori_loop(0, npairs, pair, 0)
    for p in range(2):
      drain(gpc, ssem[p], p, True)

    rem = gcnt - npairs * period
    done = 0
    for b in tail_sizes:
      pred = rem - done >= b

      @pl.when(pred)
      def _():
        g = base + npairs * period + done
        pltpu.sync_copy(eg.at[:, pl.ds(g, b)], sd_v[0].at[:, pl.ds(0, b)])
        for j in range(b):
          pltpu.async_copy(tab.at[sd_v[0].at[0, j]],
                           rows_v[0].at[pl.ds(j * LANES, LANES)], gsem[0])
        drain(b, gsem[0], 0, False)
        for j in range(b):
          pltpu.sync_copy(rows_v[0].at[pl.ds(j * LANES, LANES)],
                          acc.at[sd_v[0].at[1, j]], add=True)

      done = jnp.where(pred, done + b, done)

    plsc.subcore_barrier()
    pltpu.sync_copy(acc.at[pl.ds(r0, rpt)],
                    out.at[pl.ds(r0, rpt), pl.ds(c * feat, feat)])

  return seg_sum


def _mlp1_body(a, W1a, b1a, W1b, b1b, h1):
  x = a[:, :W1a.shape[0]]
  t = jnp.maximum(jnp.dot(x, W1a[...]) + b1a[...], 0.0)
  h = jnp.maximum(jnp.dot(t, W1b[...]) + b1b[...], 0.0)
  h1[...] = jnp.pad(h, ((0, 0), (0, 128 - h.shape[1])))


def _mlp2_body(a, W2a, b2a, W2b, b2b, Wr1, br1, Wr2, br2,
               peW, peb, hW, hb, out):
  x = a[:, :W2a.shape[0]]
  h = jnp.dot(jnp.maximum(jnp.dot(x, W2a[...]) + b2a[...], 0.0),
              W2b[...]) + b2b[...]
  pe = jnp.dot(jnp.maximum(jnp.dot(h, Wr1[...]) + br1[...], 0.0),
               Wr2[...]) + br2[...]
  emb = jnp.dot(pe, peW[...]) + peb[...]
  out[...] = jnp.dot(emb, hW[...]) + hb[...]


def _full(i):
  return (0, 0)


def kernel(W, edge_index, W1a, b1a, W1b, b1b, W2a, b2a, W2b, b2b,
           Wr1, br1, Wr2, br2, pe_W, pe_b, head_W, head_b):
  n, m = W.shape
  e = edge_index.shape[1]
  hid = W1b.shape[1]
  pe_dims = W2b.shape[1]
  channels = pe_W.shape[1]
  out_dim = head_W.shape[1]

  rpt = -(-(n + 1) // NS)
  rpt += (-rpt) % 8
  n_pad = NS * rpt
  m_half = 16 * (-(-(-(-m // 2)) // 16))

  egrp = e // LANES
  assert egrp * LANES == e, "edge count must be lane-aligned"
  eg = edge_index.reshape(2, egrp, LANES)

  w128 = jnp.pad(W, ((0, n_pad - n), (0, 128 - m)))

  (agg1,) = _seg_sum_sc(n_pad, egrp, m_half, 6, (6, 4, 2, 1), True)(
      w128, eg)

  nb = -(-n // BR)
  W1a_p = jnp.pad(W1a, ((0, 2 * m_half - m), (0, 0)))
  h1 = pl.pallas_call(
      _mlp1_body,
      grid=(nb,),
      in_specs=[
          pl.BlockSpec((BR, 128), lambda i: (i, 0)),
          pl.BlockSpec((2 * m_half, hid), _full),
          pl.BlockSpec((1, hid), _full),
          pl.BlockSpec((hid, hid), _full),
          pl.BlockSpec((1, hid), _full),
      ],
      out_specs=pl.BlockSpec((BR, 128), lambda i: (i, 0)),
      out_shape=jax.ShapeDtypeStruct((n_pad, 128), jnp.float32),
  )(agg1, W1a_p, b1a.reshape(1, hid), W1b, b1b.reshape(1, hid))

  agg2, _ = _seg_sum_sc(n_pad, egrp, hid // 2, 3, (2, 2, 1), False)(h1, eg)

  out = pl.pallas_call(
      _mlp2_body,
      grid=(nb,),
      in_specs=[
          pl.BlockSpec((BR, 128), lambda i: (i, 0)),
          pl.BlockSpec((hid, hid), _full),
          pl.BlockSpec((1, hid), _full),
          pl.BlockSpec((hid, pe_dims), _full),
          pl.BlockSpec((1, pe_dims), _full),
          pl.BlockSpec((pe_dims, hid), _full),
          pl.BlockSpec((1, hid), _full),
          pl.BlockSpec((hid, pe_dims), _full),
          pl.BlockSpec((1, pe_dims), _full),
          pl.BlockSpec((pe_dims, channels), _full),
          pl.BlockSpec((1, channels), _full),
          pl.BlockSpec((channels, out_dim), _full),
          pl.BlockSpec((1, out_dim), _full),
      ],
      out_specs=pl.BlockSpec((BR, out_dim), lambda i: (i, 0)),
      out_shape=jax.ShapeDtypeStruct((nb * BR, out_dim), jnp.float32),
  )(agg2, W2a, b2a.reshape(1, hid), W2b, b2b.reshape(1, pe_dims),
    Wr1, br1.reshape(1, hid), Wr2, br2.reshape(1, pe_dims),
    pe_W, pe_b.reshape(1, channels), head_W, head_b.reshape(1, out_dim))

  return out[:n]


# speedup vs baseline: 1.0982x; 1.0982x over previous
import functools

import jax
import jax.numpy as jnp
from jax import lax
from jax.experimental import pallas as pl
from jax.experimental.pallas import tpu as pltpu
from jax.experimental.pallas import tpu_sc as plsc

NC = 2
NS = 16
LANES = 128

BR = 2000


def _seg_sum_sc(n_pad, e_groups, feat, gpc, tail_sizes, spmem_table):
  rpt = n_pad // NS
  gfloor = e_groups // NS
  grem = e_groups - NS * gfloor
  period = 2 * gpc
  assert sum(tail_sizes) >= period - 1 and max(tail_sizes) <= gpc

  mesh = plsc.VectorSubcoreMesh(core_axis_name="c", subcore_axis_name="s")

  out_type = [jax.ShapeDtypeStruct((n_pad, 128), jnp.float32)]
  scratch = [
      pltpu.VMEM_SHARED((n_pad, feat), jnp.float32),
      [pltpu.VMEM((2, gpc, LANES), jnp.int32)] * 2,
      [pltpu.VMEM((gpc * LANES, feat), jnp.float32)] * 2,
      [pltpu.SemaphoreType.DMA] * 2,
      [pltpu.SemaphoreType.DMA] * 2,
  ]
  if spmem_table:
    scratch.append(pltpu.VMEM_SHARED((n_pad, feat), jnp.float32))
  else:
    out_type.append(jax.ShapeDtypeStruct((NC, n_pad, feat), jnp.float32))

  @functools.partial(
      pl.kernel,
      out_type=tuple(out_type),
      mesh=mesh,
      scratch_types=scratch,
      compiler_params=pltpu.CompilerParams(use_tc_tiling_on_sc=False),
  )
  def seg_sum(tabx, eg, out, *rest):
    if spmem_table:
      acc, sd_v, rows_v, gsem, ssem, stab = rest
    else:
      tabh, acc, sd_v, rows_v, gsem, ssem = rest
    c = lax.axis_index("c")
    s = lax.axis_index("s")
    r0 = s * rpt
    pltpu.sync_copy(tabx.at[pl.ds(r0, rpt), pl.ds(c * feat, feat)],
                    acc.at[pl.ds(r0, rpt)])
    if spmem_table:
      tab = stab
      pltpu.sync_copy(tabx.at[pl.ds(r0, rpt), pl.ds(c * feat, feat)],
                      stab.at[pl.ds(r0, rpt)])
    else:
      tab = tabh.at[c]
      pltpu.sync_copy(acc.at[pl.ds(r0, rpt)], tabh.at[c, pl.ds(r0, rpt)])
    plsc.subcore_barrier()

    base = s * gfloor + jnp.minimum(s, grem)
    gcnt = gfloor + jnp.where(s < grem, 1, 0)
    npairs = gcnt // period

    def drain(n_groups, sem, p, to_acc):
      nr = n_groups * LANES
      if to_acc:
        pltpu.make_async_copy(rows_v[p].at[pl.ds(0, nr)],
                              acc.at[pl.ds(0, nr)], sem).wait()
      else:
        pltpu.make_async_copy(tab.at[pl.ds(0, nr)],
                              rows_v[p].at[pl.ds(0, nr)], sem).wait()

    def fire_gathers(p, g0):
      pltpu.sync_copy(eg.at[:, pl.ds(g0, gpc)], sd_v[p])
      for j in range(gpc):
        pltpu.async_copy(tab.at[sd_v[p].at[0, j]],
                         rows_v[p].at[pl.ds(j * LANES, LANES)], gsem[p])

    for p in range(2):
      fire_gathers(p, base + p * gpc)

    def pair(k, carry):
      for p in range(2):
        drain(gpc, gsem[p], p, False)
        for j in range(gpc):
          pltpu.async_copy(rows_v[p].at[pl.ds(j * LANES, LANES)],
                           acc.at[sd_v[p].at[1, j]], ssem[p], add=True)

        @pl.when(k < npairs - 1)
        def _():
          drain(gpc, ssem[p], p, True)
          fire_gathers(p, base + (2 * k + p + 2) * gpc)
      return carry

    lax.fori_loop(0, npairs, pair, 0)
    for p in range(2):
      drain(gpc, ssem[p], p, True)

    rem = gcnt - npairs * period
    done = 0
    for b in tail_sizes:
      pred = rem - done >= b

      @pl.when(pred)
      def _():
        g = base + npairs * period + done
        pltpu.sync_copy(eg.at[:, pl.ds(g, b)], sd_v[0].at[:, pl.ds(0, b)])
        for j in range(b):
          pltpu.async_copy(tab.at[sd_v[0].at[0, j]],
                           rows_v[0].at[pl.ds(j * LANES, LANES)], gsem[0])
        drain(b, gsem[0], 0, False)
        for j in range(b):
          pltpu.sync_copy(rows_v[0].at[pl.ds(j * LANES, LANES)],
                          acc.at[sd_v[0].at[1, j]], add=True)

      done = jnp.where(pred, done + b, done)

    plsc.subcore_barrier()
    pltpu.sync_copy(acc.at[pl.ds(r0, rpt)],
                    out.at[pl.ds(r0, rpt), pl.ds(c * feat, feat)])

  return seg_sum


def _mlp1_body(a, W1a, b1a, W1b, b1b, h1):
  x = a[:, :W1a.shape[0]]
  t = jnp.maximum(jnp.dot(x, W1a[...]) + b1a[...], 0.0)
  h = jnp.maximum(jnp.dot(t, W1b[...]) + b1b[...], 0.0)
  h1[...] = jnp.pad(h, ((0, 0), (0, 128 - h.shape[1])))


def _mlp2_body(a, W2a, b2a, W2b, b2b, Wr1, br1, Wr2, br2,
               peW, peb, hW, hb, out):
  x = a[:, :W2a.shape[0]]
  h = jnp.dot(jnp.maximum(jnp.dot(x, W2a[...]) + b2a[...], 0.0),
              W2b[...]) + b2b[...]
  pe = jnp.dot(jnp.maximum(jnp.dot(h, Wr1[...]) + br1[...], 0.0),
               Wr2[...]) + br2[...]
  emb = jnp.dot(pe, peW[...]) + peb[...]
  out[...] = jnp.dot(emb, hW[...]) + hb[...]


def _full(i):
  return (0, 0)


def kernel(W, edge_index, W1a, b1a, W1b, b1b, W2a, b2a, W2b, b2b,
           Wr1, br1, Wr2, br2, pe_W, pe_b, head_W, head_b):
  n, m = W.shape
  e = edge_index.shape[1]
  hid = W1b.shape[1]
  pe_dims = W2b.shape[1]
  channels = pe_W.shape[1]
  out_dim = head_W.shape[1]

  rpt = -(-(n + 1) // NS)
  rpt += (-rpt) % 8
  n_pad = NS * rpt
  m_half = 16 * (-(-(-(-m // 2)) // 16))

  egrp = e // LANES
  assert egrp * LANES == e, "edge count must be lane-aligned"
  eg = edge_index.reshape(2, egrp, LANES)

  w128 = jnp.pad(W, ((0, n_pad - n), (0, 128 - m)))

  agg1, _ = _seg_sum_sc(n_pad, egrp, m_half, 8, (8, 4, 2, 1), False)(
      w128, eg)

  nb = -(-n // BR)
  W1a_p = jnp.pad(W1a, ((0, 2 * m_half - m), (0, 0)))
  h1 = pl.pallas_call(
      _mlp1_body,
      grid=(nb,),
      in_specs=[
          pl.BlockSpec((BR, 128), lambda i: (i, 0)),
          pl.BlockSpec((2 * m_half, hid), _full),
          pl.BlockSpec((1, hid), _full),
          pl.BlockSpec((hid, hid), _full),
          pl.BlockSpec((1, hid), _full),
      ],
      out_specs=pl.BlockSpec((BR, 128), lambda i: (i, 0)),
      out_shape=jax.ShapeDtypeStruct((n_pad, 128), jnp.float32),
  )(agg1, W1a_p, b1a.reshape(1, hid), W1b, b1b.reshape(1, hid))

  agg2, _ = _seg_sum_sc(n_pad, egrp, hid // 2, 3, (2, 2, 1), False)(h1, eg)

  out = pl.pallas_call(
      _mlp2_body,
      grid=(nb,),
      in_specs=[
          pl.BlockSpec((BR, 128), lambda i: (i, 0)),
          pl.BlockSpec((hid, hid), _full),
          pl.BlockSpec((1, hid), _full),
          pl.BlockSpec((hid, pe_dims), _full),
          pl.BlockSpec((1, pe_dims), _full),
          pl.BlockSpec((pe_dims, hid), _full),
          pl.BlockSpec((1, hid), _full),
          pl.BlockSpec((hid, pe_dims), _full),
          pl.BlockSpec((1, pe_dims), _full),
          pl.BlockSpec((pe_dims, channels), _full),
          pl.BlockSpec((1, channels), _full),
          pl.BlockSpec((channels, out_dim), _full),
          pl.BlockSpec((1, out_dim), _full),
      ],
      out_specs=pl.BlockSpec((BR, out_dim), lambda i: (i, 0)),
      out_shape=jax.ShapeDtypeStruct((nb * BR, out_dim), jnp.float32),
  )(agg2, W2a, b2a.reshape(1, hid), W2b, b2b.reshape(1, pe_dims),
    Wr1, br1.reshape(1, hid), Wr2, br2.reshape(1, pe_dims),
    pe_W, pe_b.reshape(1, channels), head_W, head_b.reshape(1, out_dim))

  return out[:n]


# speedup vs baseline: 1.1827x; 1.0769x over previous
import functools

import jax
import jax.numpy as jnp
from jax import lax
from jax.experimental import pallas as pl
from jax.experimental.pallas import tpu as pltpu
from jax.experimental.pallas import tpu_sc as plsc

NC = 2
NS = 16
LANES = 128

BR = 2000


def _seg_sum_sc(n_pad, e_groups, feat, gpc, tail_sizes, spmem_table):
  rpt = n_pad // NS
  gfloor = e_groups // NS
  grem = e_groups - NS * gfloor
  period = 2 * gpc
  assert sum(tail_sizes) >= period and max(tail_sizes) <= gpc
  assert gfloor >= 2 * period

  mesh = plsc.VectorSubcoreMesh(core_axis_name="c", subcore_axis_name="s")

  out_type = [jax.ShapeDtypeStruct((n_pad, 128), jnp.float32)]
  scratch = [
      pltpu.VMEM_SHARED((n_pad, feat), jnp.float32),
      [pltpu.VMEM((2, period, LANES), jnp.int32)] * 2,
      [pltpu.VMEM((gpc * LANES, feat), jnp.float32)] * 2,
      [pltpu.SemaphoreType.DMA] * 2,
      [pltpu.SemaphoreType.DMA] * 2,
  ]
  if spmem_table:
    scratch.append(pltpu.VMEM_SHARED((n_pad, feat), jnp.float32))
  else:
    out_type.append(jax.ShapeDtypeStruct((NC, n_pad, feat), jnp.float32))

  @functools.partial(
      pl.kernel,
      out_type=tuple(out_type),
      mesh=mesh,
      scratch_types=scratch,
      compiler_params=pltpu.CompilerParams(use_tc_tiling_on_sc=False),
  )
  def seg_sum(tabx, eg, out, *rest):
    if spmem_table:
      acc, sd_v, rows_v, gsem, ssem, stab = rest
    else:
      tabh, acc, sd_v, rows_v, gsem, ssem = rest
    c = lax.axis_index("c")
    s = lax.axis_index("s")
    r0 = s * rpt
    pltpu.sync_copy(tabx.at[pl.ds(r0, rpt), pl.ds(c * feat, feat)],
                    acc.at[pl.ds(r0, rpt)])
    if spmem_table:
      tab = stab
      pltpu.sync_copy(tabx.at[pl.ds(r0, rpt), pl.ds(c * feat, feat)],
                      stab.at[pl.ds(r0, rpt)])
    else:
      tab = tabh.at[c]
      pltpu.sync_copy(acc.at[pl.ds(r0, rpt)], tabh.at[c, pl.ds(r0, rpt)])
    plsc.subcore_barrier()

    base = s * gfloor + jnp.minimum(s, grem)
    gcnt = gfloor + jnp.where(s < grem, 1, 0)
    npairs = gfloor // period

    def drain(n_groups, sem, p, to_acc):
      nr = n_groups * LANES
      if to_acc:
        pltpu.make_async_copy(rows_v[p].at[pl.ds(0, nr)],
                              acc.at[pl.ds(0, nr)], sem).wait()
      else:
        pltpu.make_async_copy(tab.at[pl.ds(0, nr)],
                              rows_v[p].at[pl.ds(0, nr)], sem).wait()

    def load_pair(q, kp):
      pltpu.sync_copy(eg.at[:, pl.ds(base + kp * period, period)], sd_v[q])

    def fire_gathers(q, p):
      for j in range(gpc):
        pltpu.async_copy(tab.at[sd_v[q].at[0, p * gpc + j]],
                         rows_v[p].at[pl.ds(j * LANES, LANES)], gsem[p])

    load_pair(0, 0)
    for p in range(2):
      fire_gathers(0, p)

    def do_pair(k, q, qn, more):
      @pl.when(more)
      def _():
        load_pair(qn, k + 1)
      for p in range(2):
        drain(gpc, gsem[p], p, False)
        for j in range(gpc):
          pltpu.async_copy(rows_v[p].at[pl.ds(j * LANES, LANES)],
                           acc.at[sd_v[q].at[1, p * gpc + j]], ssem[p],
                           add=True)

        @pl.when(more)
        def _():
          drain(gpc, ssem[p], p, True)
          fire_gathers(qn, p)

    def superpair(u, carry):
      do_pair(2 * u, 0, 1, 2 * u < npairs - 1)
      do_pair(2 * u + 1, 1, 0, 2 * u + 1 < npairs - 1)
      return carry

    lax.fori_loop(0, npairs // 2, superpair, 0)
    if npairs % 2:
      for p in range(2):
        drain(gpc, gsem[p], p, False)
        for j in range(gpc):
          pltpu.async_copy(rows_v[p].at[pl.ds(j * LANES, LANES)],
                           acc.at[sd_v[0].at[1, p * gpc + j]], ssem[p],
                           add=True)
    for p in range(2):
      drain(gpc, ssem[p], p, True)

    rem = gcnt - npairs * period
    done = 0
    for b in tail_sizes:
      pred = rem - done >= b

      @pl.when(pred)
      def _():
        g = base + npairs * period + done
        pltpu.sync_copy(eg.at[:, pl.ds(g, b)], sd_v[0].at[:, pl.ds(0, b)])
        for j in range(b):
          pltpu.async_copy(tab.at[sd_v[0].at[0, j]],
                           rows_v[0].at[pl.ds(j * LANES, LANES)], gsem[0])
        drain(b, gsem[0], 0, False)
        for j in range(b):
          pltpu.sync_copy(rows_v[0].at[pl.ds(j * LANES, LANES)],
                          acc.at[sd_v[0].at[1, j]], add=True)

      done = jnp.where(pred, done + b, done)

    plsc.subcore_barrier()
    pltpu.sync_copy(acc.at[pl.ds(r0, rpt)],
                    out.at[pl.ds(r0, rpt), pl.ds(c * feat, feat)])

  return seg_sum


def _mlp1_body(a, W1a, b1a, W1b, b1b, h1):
  x = a[:, :W1a.shape[0]]
  t = jnp.maximum(jnp.dot(x, W1a[...]) + b1a[...], 0.0)
  h = jnp.maximum(jnp.dot(t, W1b[...]) + b1b[...], 0.0)
  h1[...] = jnp.pad(h, ((0, 0), (0, 128 - h.shape[1])))


def _mlp2_body(a, W2a, b2a, W2b, b2b, Wr1, br1, Wr2, br2,
               peW, peb, hW, hb, out):
  x = a[:, :W2a.shape[0]]
  h = jnp.dot(jnp.maximum(jnp.dot(x, W2a[...]) + b2a[...], 0.0),
              W2b[...]) + b2b[...]
  pe = jnp.dot(jnp.maximum(jnp.dot(h, Wr1[...]) + br1[...], 0.0),
               Wr2[...]) + br2[...]
  emb = jnp.dot(pe, peW[...]) + peb[...]
  out[...] = jnp.dot(emb, hW[...]) + hb[...]


def _full(i):
  return (0, 0)


def kernel(W, edge_index, W1a, b1a, W1b, b1b, W2a, b2a, W2b, b2b,
           Wr1, br1, Wr2, br2, pe_W, pe_b, head_W, head_b):
  n, m = W.shape
  e = edge_index.shape[1]
  hid = W1b.shape[1]
  pe_dims = W2b.shape[1]
  channels = pe_W.shape[1]
  out_dim = head_W.shape[1]

  rpt = -(-(n + 1) // NS)
  rpt += (-rpt) % 8
  n_pad = NS * rpt
  m_half = 16 * (-(-(-(-m // 2)) // 16))

  egrp = e // LANES
  assert egrp * LANES == e, "edge count must be lane-aligned"
  eg = edge_index.reshape(2, egrp, LANES)

  w128 = jnp.pad(W, ((0, n_pad - n), (0, 128 - m)))

  agg1, _ = _seg_sum_sc(n_pad, egrp, m_half, 8, (8, 8, 4, 2, 1), False)(
      w128, eg)

  nb = -(-n // BR)
  W1a_p = jnp.pad(W1a, ((0, 2 * m_half - m), (0, 0)))
  h1 = pl.pallas_call(
      _mlp1_body,
      grid=(nb,),
      in_specs=[
          pl.BlockSpec((BR, 128), lambda i: (i, 0)),
          pl.BlockSpec((2 * m_half, hid), _full),
          pl.BlockSpec((1, hid), _full),
          pl.BlockSpec((hid, hid), _full),
          pl.BlockSpec((1, hid), _full),
      ],
      out_specs=pl.BlockSpec((BR, 128), lambda i: (i, 0)),
      out_shape=jax.ShapeDtypeStruct((n_pad, 128), jnp.float32),
  )(agg1, W1a_p, b1a.reshape(1, hid), W1b, b1b.reshape(1, hid))

  agg2, _ = _seg_sum_sc(n_pad, egrp, hid // 2, 3, (3, 3, 2, 1), False)(
      h1, eg)

  out = pl.pallas_call(
      _mlp2_body,
      grid=(nb,),
      in_specs=[
          pl.BlockSpec((BR, 128), lambda i: (i, 0)),
          pl.BlockSpec((hid, hid), _full),
          pl.BlockSpec((1, hid), _full),
          pl.BlockSpec((hid, pe_dims), _full),
          pl.BlockSpec((1, pe_dims), _full),
          pl.BlockSpec((pe_dims, hid), _full),
          pl.BlockSpec((1, hid), _full),
          pl.BlockSpec((hid, pe_dims), _full),
          pl.BlockSpec((1, pe_dims), _full),
          pl.BlockSpec((pe_dims, channels), _full),
          pl.BlockSpec((1, channels), _full),
          pl.BlockSpec((channels, out_dim), _full),
          pl.BlockSpec((1, out_dim), _full),
      ],
      out_specs=pl.BlockSpec((BR, out_dim), lambda i: (i, 0)),
      out_shape=jax.ShapeDtypeStruct((nb * BR, out_dim), jnp.float32),
  )(agg2, W2a, b2a.reshape(1, hid), W2b, b2b.reshape(1, pe_dims),
    Wr1, br1.reshape(1, hid), Wr2, br2.reshape(1, pe_dims),
    pe_W, pe_b.reshape(1, channels), head_W, head_b.reshape(1, out_dim))

  return out[:n]


# speedup vs baseline: 1.1871x; 1.0037x over previous
import functools

import jax
import jax.numpy as jnp
from jax import lax
from jax.experimental import pallas as pl
from jax.experimental.pallas import tpu as pltpu
from jax.experimental.pallas import tpu_sc as plsc

NC = 2
NS = 16
LANES = 128

BR = 2000


def _seg_sum_sc(n_pad, e_groups, feat, gpc, tail_sizes, spmem_table):
  rpt = n_pad // NS
  gfloor = e_groups // NS
  grem = e_groups - NS * gfloor
  period = 2 * gpc
  assert sum(tail_sizes) >= period and max(tail_sizes) <= gpc
  assert gfloor >= 2 * period

  mesh = plsc.VectorSubcoreMesh(core_axis_name="c", subcore_axis_name="s")

  out_type = [jax.ShapeDtypeStruct((n_pad, 128), jnp.float32)]
  scratch = [
      pltpu.VMEM_SHARED((n_pad, feat), jnp.float32),
      [pltpu.VMEM((period, 2, LANES), jnp.int32)] * 2,
      [pltpu.VMEM((gpc * LANES, feat), jnp.float32)] * 2,
      [pltpu.SemaphoreType.DMA] * 2,
      [pltpu.SemaphoreType.DMA] * 2,
  ]
  if spmem_table:
    scratch.append(pltpu.VMEM_SHARED((n_pad, feat), jnp.float32))
  else:
    out_type.append(jax.ShapeDtypeStruct((NC, n_pad, feat), jnp.float32))

  @functools.partial(
      pl.kernel,
      out_type=tuple(out_type),
      mesh=mesh,
      scratch_types=scratch,
      compiler_params=pltpu.CompilerParams(use_tc_tiling_on_sc=False),
  )
  def seg_sum(tabx, eg, out, *rest):
    if spmem_table:
      acc, sd_v, rows_v, gsem, ssem, stab = rest
    else:
      tabh, acc, sd_v, rows_v, gsem, ssem = rest
    c = lax.axis_index("c")
    s = lax.axis_index("s")
    r0 = s * rpt
    pltpu.sync_copy(tabx.at[pl.ds(r0, rpt), pl.ds(c * feat, feat)],
                    acc.at[pl.ds(r0, rpt)])
    if spmem_table:
      tab = stab
      pltpu.sync_copy(tabx.at[pl.ds(r0, rpt), pl.ds(c * feat, feat)],
                      stab.at[pl.ds(r0, rpt)])
    else:
      tab = tabh.at[c]
      pltpu.sync_copy(acc.at[pl.ds(r0, rpt)], tabh.at[c, pl.ds(r0, rpt)])
    plsc.subcore_barrier()

    base = s * gfloor + jnp.minimum(s, grem)
    gcnt = gfloor + jnp.where(s < grem, 1, 0)
    npairs = gfloor // period

    def drain(n_groups, sem, p, to_acc):
      nr = n_groups * LANES
      if to_acc:
        pltpu.make_async_copy(rows_v[p].at[pl.ds(0, nr)],
                              acc.at[pl.ds(0, nr)], sem).wait()
      else:
        pltpu.make_async_copy(tab.at[pl.ds(0, nr)],
                              rows_v[p].at[pl.ds(0, nr)], sem).wait()

    def load_pair(q, kp):
      pltpu.sync_copy(eg.at[pl.ds(base + kp * period, period)], sd_v[q])

    def fire_gathers(q, p):
      for j in range(gpc):
        pltpu.async_copy(tab.at[sd_v[q].at[p * gpc + j, 0]],
                         rows_v[p].at[pl.ds(j * LANES, LANES)], gsem[p])

    load_pair(0, 0)
    for p in range(2):
      fire_gathers(0, p)

    def do_pair(k, q, qn, more):
      @pl.when(more)
      def _():
        load_pair(qn, k + 1)
      for p in range(2):
        drain(gpc, gsem[p], p, False)
        for j in range(gpc):
          pltpu.async_copy(rows_v[p].at[pl.ds(j * LANES, LANES)],
                           acc.at[sd_v[q].at[p * gpc + j, 1]], ssem[p],
                           add=True)

        @pl.when(more)
        def _():
          drain(gpc, ssem[p], p, True)
          fire_gathers(qn, p)

    def superpair(u, carry):
      do_pair(2 * u, 0, 1, 2 * u < npairs - 1)
      do_pair(2 * u + 1, 1, 0, 2 * u + 1 < npairs - 1)
      return carry

    lax.fori_loop(0, npairs // 2, superpair, 0)
    if npairs % 2:
      for p in range(2):
        drain(gpc, gsem[p], p, False)
        for j in range(gpc):
          pltpu.async_copy(rows_v[p].at[pl.ds(j * LANES, LANES)],
                           acc.at[sd_v[0].at[p * gpc + j, 1]], ssem[p],
                           add=True)
    for p in range(2):
      drain(gpc, ssem[p], p, True)

    rem = gcnt - npairs * period
    done = 0
    for b in tail_sizes:
      pred = rem - done >= b

      @pl.when(pred)
      def _():
        g = base + npairs * period + done
        pltpu.sync_copy(eg.at[pl.ds(g, b)], sd_v[0].at[pl.ds(0, b)])
        for j in range(b):
          pltpu.async_copy(tab.at[sd_v[0].at[j, 0]],
                           rows_v[0].at[pl.ds(j * LANES, LANES)], gsem[0])
        drain(b, gsem[0], 0, False)
        for j in range(b):
          pltpu.sync_copy(rows_v[0].at[pl.ds(j * LANES, LANES)],
                          acc.at[sd_v[0].at[j, 1]], add=True)

      done = jnp.where(pred, done + b, done)

    plsc.subcore_barrier()
    pltpu.sync_copy(acc.at[pl.ds(r0, rpt)],
                    out.at[pl.ds(r0, rpt), pl.ds(c * feat, feat)])

  return seg_sum


def _mlp1_body(a, W1a, b1a, W1b, b1b, h1):
  x = a[:, :W1a.shape[0]]
  t = jnp.maximum(jnp.dot(x, W1a[...]) + b1a[...], 0.0)
  h = jnp.maximum(jnp.dot(t, W1b[...]) + b1b[...], 0.0)
  h1[...] = jnp.pad(h, ((0, 0), (0, 128 - h.shape[1])))


def _mlp2_body(a, W2a, b2a, W2b, b2b, Wr1, br1, Wr2, br2,
               peW, peb, hW, hb, out):
  x = a[:, :W2a.shape[0]]
  h = jnp.dot(jnp.maximum(jnp.dot(x, W2a[...]) + b2a[...], 0.0),
              W2b[...]) + b2b[...]
  pe = jnp.dot(jnp.maximum(jnp.dot(h, Wr1[...]) + br1[...], 0.0),
               Wr2[...]) + br2[...]
  emb = jnp.dot(pe, peW[...]) + peb[...]
  out[...] = jnp.dot(emb, hW[...]) + hb[...]


def _full(i):
  return (0, 0)


def kernel(W, edge_index, W1a, b1a, W1b, b1b, W2a, b2a, W2b, b2b,
           Wr1, br1, Wr2, br2, pe_W, pe_b, head_W, head_b):
  n, m = W.shape
  e = edge_index.shape[1]
  hid = W1b.shape[1]
  pe_dims = W2b.shape[1]
  channels = pe_W.shape[1]
  out_dim = head_W.shape[1]

  rpt = -(-(n + 1) // NS)
  rpt += (-rpt) % 8
  n_pad = NS * rpt
  m_half = 16 * (-(-(-(-m // 2)) // 16))

  egrp = e // LANES
  assert egrp * LANES == e, "edge count must be lane-aligned"
  eg = edge_index.reshape(2, egrp, LANES).transpose(1, 0, 2)

  w128 = jnp.pad(W, ((0, n_pad - n), (0, 128 - m)))

  agg1, _ = _seg_sum_sc(n_pad, egrp, m_half, 8, (8, 8, 4, 2, 1), False)(
      w128, eg)

  nb = -(-n // BR)
  W1a_p = jnp.pad(W1a, ((0, 2 * m_half - m), (0, 0)))
  h1 = pl.pallas_call(
      _mlp1_body,
      grid=(nb,),
      in_specs=[
          pl.BlockSpec((BR, 128), lambda i: (i, 0)),
          pl.BlockSpec((2 * m_half, hid), _full),
          pl.BlockSpec((1, hid), _full),
          pl.BlockSpec((hid, hid), _full),
          pl.BlockSpec((1, hid), _full),
      ],
      out_specs=pl.BlockSpec((BR, 128), lambda i: (i, 0)),
      out_shape=jax.ShapeDtypeStruct((n_pad, 128), jnp.float32),
  )(agg1, W1a_p, b1a.reshape(1, hid), W1b, b1b.reshape(1, hid))

  agg2, _ = _seg_sum_sc(n_pad, egrp, hid // 2, 3, (3, 3, 2, 1), False)(
      h1, eg)

  out = pl.pallas_call(
      _mlp2_body,
      grid=(nb,),
      in_specs=[
          pl.BlockSpec((BR, 128), lambda i: (i, 0)),
          pl.BlockSpec((hid, hid), _full),
          pl.BlockSpec((1, hid), _full),
          pl.BlockSpec((hid, pe_dims), _full),
          pl.BlockSpec((1, pe_dims), _full),
          pl.BlockSpec((pe_dims, hid), _full),
          pl.BlockSpec((1, hid), _full),
          pl.BlockSpec((hid, pe_dims), _full),
          pl.BlockSpec((1, pe_dims), _full),
          pl.BlockSpec((pe_dims, channels), _full),
          pl.BlockSpec((1, channels), _full),
          pl.BlockSpec((channels, out_dim), _full),
          pl.BlockSpec((1, out_dim), _full),
      ],
      out_specs=pl.BlockSpec((BR, out_dim), lambda i: (i, 0)),
      out_shape=jax.ShapeDtypeStruct((nb * BR, out_dim), jnp.float32),
  )(agg2, W2a, b2a.reshape(1, hid), W2b, b2b.reshape(1, pe_dims),
    Wr1, br1.reshape(1, hid), Wr2, br2.reshape(1, pe_dims),
    pe_W, pe_b.reshape(1, channels), head_W, head_b.reshape(1, out_dim))

  return out[:n]


# speedup vs baseline: 1.1986x; 1.0097x over previous
import functools

import jax
import jax.numpy as jnp
from jax import lax
from jax.experimental import pallas as pl
from jax.experimental.pallas import tpu as pltpu
from jax.experimental.pallas import tpu_sc as plsc

NC = 2
NS = 16
LANES = 128

BR = 2000


def _seg_sum_sc(n_pad, e_groups, feat, gpc, tail_sizes, spmem_table):
  rpt = n_pad // NS
  gfloor = e_groups // NS
  grem = e_groups - NS * gfloor
  period = 2 * gpc
  assert sum(tail_sizes) >= period and max(tail_sizes) <= gpc
  assert gfloor >= 2 * period

  mesh = plsc.VectorSubcoreMesh(core_axis_name="c", subcore_axis_name="s")

  out_type = [jax.ShapeDtypeStruct((n_pad, 128), jnp.float32)]
  scratch = [
      pltpu.VMEM_SHARED((n_pad, feat), jnp.float32),
      [pltpu.VMEM((period, 2, LANES), jnp.int32)] * 2,
      [pltpu.VMEM((gpc * LANES, feat), jnp.float32)] * 2,
      [pltpu.SemaphoreType.DMA] * 2,
      [pltpu.SemaphoreType.DMA] * 2,
  ]
  if spmem_table:
    scratch.append(pltpu.VMEM_SHARED((n_pad, feat), jnp.float32))
  else:
    out_type.append(jax.ShapeDtypeStruct((NC, n_pad, feat), jnp.float32))

  @functools.partial(
      pl.kernel,
      out_type=tuple(out_type),
      mesh=mesh,
      scratch_types=scratch,
      compiler_params=pltpu.CompilerParams(use_tc_tiling_on_sc=False),
  )
  def seg_sum(tabx, eg, out, *rest):
    if spmem_table:
      acc, sd_v, rows_v, gsem, ssem, stab = rest
    else:
      tabh, acc, sd_v, rows_v, gsem, ssem = rest
    c = lax.axis_index("c")
    s = lax.axis_index("s")
    r0 = s * rpt
    pltpu.sync_copy(tabx.at[pl.ds(r0, rpt), pl.ds(c * feat, feat)],
                    acc.at[pl.ds(r0, rpt)])
    if spmem_table:
      tab = stab
      pltpu.sync_copy(tabx.at[pl.ds(r0, rpt), pl.ds(c * feat, feat)],
                      stab.at[pl.ds(r0, rpt)])
    else:
      tab = tabh.at[c]
      pltpu.sync_copy(acc.at[pl.ds(r0, rpt)], tabh.at[c, pl.ds(r0, rpt)])
    plsc.subcore_barrier()

    base = s * gfloor + jnp.minimum(s, grem)
    gcnt = gfloor + jnp.where(s < grem, 1, 0)
    npairs = gfloor // period

    def drain(n_groups, sem, p, to_acc):
      nr = n_groups * LANES
      if to_acc:
        pltpu.make_async_copy(rows_v[p].at[pl.ds(0, nr)],
                              acc.at[pl.ds(0, nr)], sem).wait()
      else:
        pltpu.make_async_copy(tab.at[pl.ds(0, nr)],
                              rows_v[p].at[pl.ds(0, nr)], sem).wait()

    def load_pair(q, kp):
      pltpu.sync_copy(eg.at[pl.ds(base + kp * period, period)], sd_v[q])

    def fire_gathers(q, p):
      for j in range(gpc):
        pltpu.async_copy(tab.at[sd_v[q].at[p * gpc + j, 0]],
                         rows_v[p].at[pl.ds(j * LANES, LANES)], gsem[p])

    load_pair(0, 0)
    for p in range(2):
      fire_gathers(0, p)

    def do_pair(k, q, qn, more):
      @pl.when(more)
      def _():
        load_pair(qn, k + 1)
      for p in range(2):
        drain(gpc, gsem[p], p, False)
        for j in range(gpc):
          pltpu.async_copy(rows_v[p].at[pl.ds(j * LANES, LANES)],
                           acc.at[sd_v[q].at[p * gpc + j, 1]], ssem[p],
                           add=True)

        @pl.when(more)
        def _():
          drain(gpc, ssem[p], p, True)
          fire_gathers(qn, p)

    def superpair(u, carry):
      do_pair(2 * u, 0, 1, 2 * u < npairs - 1)
      do_pair(2 * u + 1, 1, 0, 2 * u + 1 < npairs - 1)
      return carry

    lax.fori_loop(0, npairs // 2, superpair, 0)
    if npairs % 2:
      for p in range(2):
        drain(gpc, gsem[p], p, False)
        for j in range(gpc):
          pltpu.async_copy(rows_v[p].at[pl.ds(j * LANES, LANES)],
                           acc.at[sd_v[0].at[p * gpc + j, 1]], ssem[p],
                           add=True)
    for p in range(2):
      drain(gpc, ssem[p], p, True)

    rem = gcnt - npairs * period
    done = 0
    for b in tail_sizes:
      pred = rem - done >= b

      @pl.when(pred)
      def _():
        g = base + npairs * period + done
        pltpu.sync_copy(eg.at[pl.ds(g, b)], sd_v[0].at[pl.ds(0, b)])
        for j in range(b):
          pltpu.async_copy(tab.at[sd_v[0].at[j, 0]],
                           rows_v[0].at[pl.ds(j * LANES, LANES)], gsem[0])
        drain(b, gsem[0], 0, False)
        for j in range(b):
          pltpu.sync_copy(rows_v[0].at[pl.ds(j * LANES, LANES)],
                          acc.at[sd_v[0].at[j, 1]], add=True)

      done = jnp.where(pred, done + b, done)

    plsc.subcore_barrier()
    pltpu.sync_copy(acc.at[pl.ds(r0, rpt)],
                    out.at[pl.ds(r0, rpt), pl.ds(c * feat, feat)])

  return seg_sum


def _mlp1_body(a, W1a, b1a, W1b, b1b, h1):
  x = a[:, :W1a.shape[0]]
  t = jnp.maximum(jnp.dot(x, W1a[...]) + b1a[...], 0.0)
  h = jnp.maximum(jnp.dot(t, W1b[...]) + b1b[...], 0.0)
  h1[...] = jnp.pad(h, ((0, 0), (0, 128 - h.shape[1])))


def _mlp2_body(a, W2a, b2a, W2b, b2b, Wr1, br1, Wr2, br2,
               peW, peb, hW, hb, out):
  x = a[:, :W2a.shape[0]]
  h = jnp.dot(jnp.maximum(jnp.dot(x, W2a[...]) + b2a[...], 0.0),
              W2b[...]) + b2b[...]
  pe = jnp.dot(jnp.maximum(jnp.dot(h, Wr1[...]) + br1[...], 0.0),
               Wr2[...]) + br2[...]
  emb = jnp.dot(pe, peW[...]) + peb[...]
  out[...] = jnp.dot(emb, hW[...]) + hb[...]


def _full(i):
  return (0, 0)


def kernel(W, edge_index, W1a, b1a, W1b, b1b, W2a, b2a, W2b, b2b,
           Wr1, br1, Wr2, br2, pe_W, pe_b, head_W, head_b):
  n, m = W.shape
  e = edge_index.shape[1]
  hid = W1b.shape[1]
  pe_dims = W2b.shape[1]
  channels = pe_W.shape[1]
  out_dim = head_W.shape[1]

  rpt = -(-(n + 1) // NS)
  rpt += (-rpt) % 8
  n_pad = NS * rpt
  m_half = 16 * (-(-(-(-m // 2)) // 16))

  egrp = e // LANES
  assert egrp * LANES == e, "edge count must be lane-aligned"
  eg = edge_index.reshape(2, egrp, LANES).transpose(1, 0, 2)

  w128 = jnp.pad(W, ((0, n_pad - n), (0, 128 - m)))

  agg1, _ = _seg_sum_sc(n_pad, egrp, m_half, 12, (12, 12, 6, 3, 2, 1),
                        False)(w128, eg)

  nb = -(-n // BR)
  W1a_p = jnp.pad(W1a, ((0, 2 * m_half - m), (0, 0)))
  h1 = pl.pallas_call(
      _mlp1_body,
      grid=(nb,),
      in_specs=[
          pl.BlockSpec((BR, 128), lambda i: (i, 0)),
          pl.BlockSpec((2 * m_half, hid), _full),
          pl.BlockSpec((1, hid), _full),
          pl.BlockSpec((hid, hid), _full),
          pl.BlockSpec((1, hid), _full),
      ],
      out_specs=pl.BlockSpec((BR, 128), lambda i: (i, 0)),
      out_shape=jax.ShapeDtypeStruct((n_pad, 128), jnp.float32),
  )(agg1, W1a_p, b1a.reshape(1, hid), W1b, b1b.reshape(1, hid))

  agg2, _ = _seg_sum_sc(n_pad, egrp, hid // 2, 3, (3, 3, 2, 1), False)(
      h1, eg)

  out = pl.pallas_call(
      _mlp2_body,
      grid=(nb,),
      in_specs=[
          pl.BlockSpec((BR, 128), lambda i: (i, 0)),
          pl.BlockSpec((hid, hid), _full),
          pl.BlockSpec((1, hid), _full),
          pl.BlockSpec((hid, pe_dims), _full),
          pl.BlockSpec((1, pe_dims), _full),
          pl.BlockSpec((pe_dims, hid), _full),
          pl.BlockSpec((1, hid), _full),
          pl.BlockSpec((hid, pe_dims), _full),
          pl.BlockSpec((1, pe_dims), _full),
          pl.BlockSpec((pe_dims, channels), _full),
          pl.BlockSpec((1, channels), _full),
          pl.BlockSpec((channels, out_dim), _full),
          pl.BlockSpec((1, out_dim), _full),
      ],
      out_specs=pl.BlockSpec((BR, out_dim), lambda i: (i, 0)),
      out_shape=jax.ShapeDtypeStruct((nb * BR, out_dim), jnp.float32),
  )(agg2, W2a, b2a.reshape(1, hid), W2b, b2b.reshape(1, pe_dims),
    Wr1, br1.reshape(1, hid), Wr2, br2.reshape(1, pe_dims),
    pe_W, pe_b.reshape(1, channels), head_W, head_b.reshape(1, out_dim))

  return out[:n]
